# Initial kernel scaffold; baseline (speedup 1.0000x reference)
import jax
import jax.numpy as jnp
from jax.experimental import pallas as pl


def kernel(x, table, W, b):
    raise NotImplementedError("write your pallas kernel here")


# speedup vs baseline: 13.4342x; 13.4342x over previous
import functools

import jax
import jax.numpy as jnp
from jax import lax
from jax.experimental import pallas as pl
from jax.experimental.pallas import tpu as pltpu
from jax.experimental.pallas import tpu_sc as plsc

B, L, D = 4096, 200, 128
NC, NS = 2, 16
NW = NC * NS
BPW = B // NW
NCHUNK = 2
CHUNK = L // NCHUNK
NLANE = 16
DV = D // NLANE


def _make_pool_body(bpw, nchunk, chunk, d):
    seq = nchunk * chunk
    dv = d // NLANE

    def body(x_hbm, table_hbm, out_hbm, idx_v, buf0, buf1, pooled_v, sem0, sem1):
        wid = lax.axis_index("s") * NC + lax.axis_index("c")
        base = wid * bpw
        pltpu.sync_copy(x_hbm.at[pl.ds(base, bpw)], idx_v)

        bufs = (buf0, buf1)
        sems = (sem0, sem1)

        def fire(r, buf, sem):
            for c in range(nchunk):
                pltpu.async_copy(
                    table_hbm.at[idx_v.at[r, c]],
                    buf.at[pl.ds(c * chunk, chunk)],
                    sem,
                )

        def wait(buf, sem):
            pltpu.make_async_copy(table_hbm.at[pl.ds(0, seq)], buf, sem).wait()

        def reduce_row(r, buf):
            def step(i, accs):
                return tuple(
                    jnp.maximum(a, buf[i, pl.ds(NLANE * k, NLANE)])
                    for k, a in enumerate(accs)
                )
            init = tuple(buf[0, pl.ds(NLANE * k, NLANE)] for k in range(dv))
            accs = lax.fori_loop(1, seq, step, init, unroll=4)
            for k in range(dv):
                pooled_v[r, pl.ds(NLANE * k, NLANE)] = accs[k]

        fire(0, bufs[0], sems[0])

        def outer(g, _):
            for s in range(2):
                r = 2 * g + s
                @pl.when(r + 1 < bpw)
                def _fire_next():
                    fire(r + 1, bufs[1 - s], sems[1 - s])
                wait(bufs[s], sems[s])
                reduce_row(r, bufs[s])
            return 0

        lax.fori_loop(0, bpw // 2, outer, 0)
        pltpu.sync_copy(pooled_v, out_hbm.at[pl.ds(base, bpw)])

    return body


def _make_pool(bsz, bpw, nchunk, chunk, d, interpret=False):
    mesh = plsc.VectorSubcoreMesh(
        core_axis_name="c", subcore_axis_name="s", num_cores=NC, num_subcores=NS
    )
    seq = nchunk * chunk
    return pl.kernel(
        _make_pool_body(bpw, nchunk, chunk, d),
        out_type=jax.ShapeDtypeStruct((bsz, d), jnp.float32),
        mesh=mesh,
        scratch_types=[
            pltpu.VMEM((bpw, nchunk, chunk), jnp.int32),
            pltpu.VMEM((seq, d), jnp.float32),
            pltpu.VMEM((seq, d), jnp.float32),
            pltpu.VMEM((bpw, d), jnp.float32),
            pltpu.SemaphoreType.DMA,
            pltpu.SemaphoreType.DMA,
        ],
        interpret=interpret,
    )


def _mm_body(p_ref, w_ref, b_ref, o_ref):
    o_ref[...] = (
        jnp.dot(p_ref[...], w_ref[...], preferred_element_type=jnp.float32)
        + b_ref[...]
    )


_BM = 512


@jax.jit
def kernel(x, table, W, b):
    xi = x.astype(jnp.int32).reshape(B, NCHUNK, CHUNK)
    pooled = _make_pool(B, BPW, NCHUNK, CHUNK, D)(xi, table)
    out = pl.pallas_call(
        _mm_body,
        grid=(B // _BM,),
        in_specs=[
            pl.BlockSpec((_BM, D), lambda i: (i, 0)),
            pl.BlockSpec((D, D), lambda i: (0, 0)),
            pl.BlockSpec((1, D), lambda i: (0, 0)),
        ],
        out_specs=pl.BlockSpec((_BM, D), lambda i: (i, 0)),
        out_shape=jax.ShapeDtypeStruct((B, D), jnp.float32),
    )(pooled, W, b.reshape(1, D))
    return out
